# baseline (speedup 1.0000x reference)
import jax
import jax.numpy as jnp
from jax.experimental import pallas as pl
from jax.experimental.pallas import tpu as pltpu


def _lane_dup_matrix(w, dtype):
    return (jnp.arange(w)[:, None] == (jnp.arange(2 * w) // 2)[None, :]).astype(dtype)


def _up2x_kernel(x_ref, e_ref, o_ref, s_ref):
    y = jnp.dot(x_ref[...], e_ref[...], preferred_element_type=jnp.float32)
    y = y.astype(o_ref.dtype)
    br, n2 = y.shape
    w = n2 // 2
    for half in range(2):
        v = y[:, half * w:(half + 1) * w]
        s_ref[half, pl.ds(0, br, 2), :] = v
        s_ref[half, pl.ds(1, br, 2), :] = v
    o_ref[:, :w] = s_ref[0]
    o_ref[:, w:] = s_ref[1]


def _up2x_rows(x2, block_rows):
    rows, w = x2.shape
    dt = x2.dtype
    e = _lane_dup_matrix(w, dt)
    br = min(block_rows, rows)
    return pl.pallas_call(
        _up2x_kernel,
        out_shape=jax.ShapeDtypeStruct((2 * rows, 2 * w), dt),
        grid=(pl.cdiv(rows, br),),
        in_specs=[
            pl.BlockSpec((br, w), lambda i: (i, 0)),
            pl.BlockSpec((w, 2 * w), lambda i: (0, 0)),
        ],
        out_specs=pl.BlockSpec((2 * br, 2 * w), lambda i: (i, 0)),
        scratch_shapes=[pltpu.VMEM((2, 2 * br, w), dt)],
        compiler_params=pltpu.CompilerParams(
            dimension_semantics=("parallel",),
            vmem_limit_bytes=60 << 20,
        ),
    )(x2, e)


def kernel(x):
    n, c, h, w = x.shape
    x2 = x.reshape(n * c * h, w)
    out2 = _up2x_rows(x2, 4096)
    return out2.reshape(n, c, 2 * h, 2 * w)


# speedup vs baseline: 1.0216x; 1.0216x over previous
import jax
import jax.numpy as jnp
from jax.experimental import pallas as pl
from jax.experimental.pallas import tpu as pltpu


def _lane_dup_matrix(w, dtype):
    return (jnp.arange(w)[:, None] == (jnp.arange(2 * w) // 2)[None, :]).astype(dtype)


def _up2x_kernel(x_ref, e_ref, o_ref, s_ref):
    y = jnp.dot(x_ref[...], e_ref[...], preferred_element_type=jnp.float32)
    y = y.astype(o_ref.dtype)
    br, n2 = y.shape
    w = n2 // 2
    for half in range(2):
        v = y[:, half * w:(half + 1) * w]
        s_ref[half, pl.ds(0, br, 2), :] = v
        s_ref[half, pl.ds(1, br, 2), :] = v
    o_ref[:, :w] = s_ref[0]
    o_ref[:, w:] = s_ref[1]


def _up2x_rows(x2, block_rows):
    rows, w = x2.shape
    dt = x2.dtype
    e = _lane_dup_matrix(w, dt)
    br = min(block_rows, rows)
    return pl.pallas_call(
        _up2x_kernel,
        out_shape=jax.ShapeDtypeStruct((2 * rows, 2 * w), dt),
        grid=(pl.cdiv(rows, br),),
        in_specs=[
            pl.BlockSpec((br, w), lambda i: (i, 0)),
            pl.BlockSpec((w, 2 * w), lambda i: (0, 0)),
        ],
        out_specs=pl.BlockSpec((2 * br, 2 * w), lambda i: (i, 0)),
        scratch_shapes=[pltpu.VMEM((2, 2 * br, w), dt)],
        compiler_params=pltpu.CompilerParams(
            dimension_semantics=("parallel",),
            vmem_limit_bytes=60 << 20,
        ),
    )(x2, e)


def kernel(x):
    n, c, h, w = x.shape
    x2 = x.reshape(n * c * h, w)
    out2 = _up2x_rows(x2, 8192)
    return out2.reshape(n, c, 2 * h, 2 * w)
